# baseline (speedup 1.0000x reference)
import functools

import jax
import jax.numpy as jnp
import numpy as np
from jax.experimental import pallas as pl
from jax.experimental.pallas import tpu as pltpu

_LIPS = np.array([61, 146, 91, 181, 84, 17, 314, 405, 321, 375,
                  78, 191, 80, 81, 82, 13, 312, 311, 310, 415,
                  95, 88, 178, 87, 14, 317, 402, 318, 324, 308,
                  291, 185, 40, 39, 37, 0, 267, 269, 270, 409], dtype=np.int64)

_NFRAMES = 16384
_NLM = 543
_NOUT = 164
_T_BLK = 16384
_GRID_T = _NFRAMES // _T_BLK

_IDX82 = np.concatenate([np.arange(468, 489), np.arange(522, 543), _LIPS])
_ROWS = [(m % 2, int(_IDX82[m // 2])) for m in range(_NOUT)]


def _gather_body(ft_hbm, out_ref, scratch_ref, sem_ref):
    i = pl.program_id(0)

    def copies(slot, chunk):
        return [
            pltpu.make_async_copy(
                ft_hbm.at[c, pl.ds(l, 1), pl.ds(chunk * _T_BLK, _T_BLK)],
                scratch_ref.at[slot, pl.ds(m, 1), :],
                sem_ref.at[slot],
            )
            for m, (c, l) in enumerate(_ROWS)
        ]

    @pl.when(i == 0)
    def _():
        for cp in copies(0, 0):
            cp.start()

    @pl.when(i + 1 < _GRID_T)
    def _():
        for cp in copies((i + 1) % 2, i + 1):
            cp.start()

    slot = i % 2
    for cp in copies(slot, i):
        cp.wait()

    x = scratch_ref[slot]
    out_ref[...] = jnp.where(jnp.isnan(x), 0.0, x)


@functools.cache
def _make_tc_gather():
    return pl.pallas_call(
        _gather_body,
        grid=(_GRID_T,),
        in_specs=[pl.BlockSpec(memory_space=pl.ANY)],
        out_specs=pl.BlockSpec((_NOUT, _T_BLK), lambda i: (0, i)),
        out_shape=jax.ShapeDtypeStruct((_NOUT, _NFRAMES), jnp.float32),
        scratch_shapes=[
            pltpu.VMEM((2, _NOUT, _T_BLK), jnp.float32),
            pltpu.SemaphoreType.DMA((2,)),
        ],
        compiler_params=pltpu.CompilerParams(
            dimension_semantics=("arbitrary",),
        ),
    )


def kernel(frames):
    ft = frames.transpose(2, 1, 0)
    out = _make_tc_gather()(ft)
    return out.T


# speedup vs baseline: 1.1546x; 1.1546x over previous
import functools

import jax
import jax.numpy as jnp
import numpy as np
from jax.experimental import pallas as pl
from jax.experimental.pallas import tpu as pltpu

_LIPS = np.array([61, 146, 91, 181, 84, 17, 314, 405, 321, 375,
                  78, 191, 80, 81, 82, 13, 312, 311, 310, 415,
                  95, 88, 178, 87, 14, 317, 402, 318, 324, 308,
                  291, 185, 40, 39, 37, 0, 267, 269, 270, 409], dtype=np.int64)

_NFRAMES = 16384
_NLM = 543
_NOUT = 164
_T_BLK = 8192
_GRID_T = _NFRAMES // _T_BLK

_IDX82 = np.concatenate([np.arange(468, 489), np.arange(522, 543), _LIPS])
_ROWS = [(m % 2, int(_IDX82[m // 2])) for m in range(_NOUT)]


def _gather_body(ft_hbm, out_ref, scratch_ref, sem_ref):
    i = pl.program_id(0)

    def copies(slot, chunk):
        return [
            pltpu.make_async_copy(
                ft_hbm.at[c, pl.ds(l, 1), pl.ds(chunk * _T_BLK, _T_BLK)],
                scratch_ref.at[slot, pl.ds(m, 1), :],
                sem_ref.at[slot],
            )
            for m, (c, l) in enumerate(_ROWS)
        ]

    @pl.when(i == 0)
    def _():
        for cp in copies(0, 0):
            cp.start()

    @pl.when(i + 1 < _GRID_T)
    def _():
        for cp in copies((i + 1) % 2, i + 1):
            cp.start()

    slot = i % 2
    cps = copies(slot, i)
    for q in range(4):
        for cp in cps[q * 41:(q + 1) * 41]:
            cp.wait()
        x = scratch_ref[slot, pl.ds(q * 41, 41), :]
        out_ref[pl.ds(q * 41, 41), :] = jnp.where(jnp.isnan(x), 0.0, x)


@functools.cache
def _make_tc_gather():
    return pl.pallas_call(
        _gather_body,
        grid=(_GRID_T,),
        in_specs=[pl.BlockSpec(memory_space=pl.ANY)],
        out_specs=pl.BlockSpec((_NOUT, _T_BLK), lambda i: (0, i)),
        out_shape=jax.ShapeDtypeStruct((_NOUT, _NFRAMES), jnp.float32),
        scratch_shapes=[
            pltpu.VMEM((2, _NOUT, _T_BLK), jnp.float32),
            pltpu.SemaphoreType.DMA((2,)),
        ],
        compiler_params=pltpu.CompilerParams(
            dimension_semantics=("arbitrary",),
        ),
    )


def kernel(frames):
    ft = frames.transpose(2, 1, 0)
    out = _make_tc_gather()(ft)
    return out.T


# speedup vs baseline: 1.1957x; 1.0356x over previous
import functools

import jax
import jax.numpy as jnp
import numpy as np
from jax.experimental import pallas as pl
from jax.experimental.pallas import tpu as pltpu

_LIPS = np.array([61, 146, 91, 181, 84, 17, 314, 405, 321, 375,
                  78, 191, 80, 81, 82, 13, 312, 311, 310, 415,
                  95, 88, 178, 87, 14, 317, 402, 318, 324, 308,
                  291, 185, 40, 39, 37, 0, 267, 269, 270, 409], dtype=np.int64)

_NFRAMES = 16384
_NLM = 543
_NOUT = 164
_T_BLK = 8192
_GRID_T = _NFRAMES // _T_BLK

_IDX82 = np.concatenate([np.arange(468, 489), np.arange(522, 543), _LIPS])
_ROWS = [(m % 2, int(_IDX82[m // 2])) for m in range(_NOUT)]


def _gather_body(ft_hbm, out_ref, scratch_ref, sem_ref):
    i = pl.program_id(0)

    def copies(slot, chunk):
        return [
            pltpu.make_async_copy(
                ft_hbm.at[c, pl.ds(l, 1), pl.ds(chunk * _T_BLK, _T_BLK)],
                scratch_ref.at[slot, pl.ds(m, 1), :],
                sem_ref.at[slot],
            )
            for m, (c, l) in enumerate(_ROWS)
        ]

    @pl.when(i == 0)
    def _():
        for cp in copies(0, 0):
            cp.start()

    @pl.when(i + 1 < _GRID_T)
    def _():
        for cp in copies((i + 1) % 2, i + 1):
            cp.start()

    slot = i % 2
    for cp in copies(slot, i):
        cp.wait()

    x = scratch_ref[slot]
    out_ref[...] = jnp.where(jnp.isnan(x), 0.0, x)


@functools.cache
def _make_tc_gather():
    return pl.pallas_call(
        _gather_body,
        grid=(_GRID_T,),
        in_specs=[pl.BlockSpec(memory_space=pl.ANY)],
        out_specs=pl.BlockSpec((_NOUT, _T_BLK), lambda i: (0, i)),
        out_shape=jax.ShapeDtypeStruct((_NOUT, _NFRAMES), jnp.float32),
        scratch_shapes=[
            pltpu.VMEM((2, _NOUT, _T_BLK), jnp.float32),
            pltpu.SemaphoreType.DMA((2,)),
        ],
        compiler_params=pltpu.CompilerParams(
            dimension_semantics=("arbitrary",),
        ),
    )


def kernel(frames):
    ft = frames.transpose(2, 1, 0)
    out = _make_tc_gather()(ft)
    return out.T
